# initial kernel scaffold (unmeasured)
import jax
import jax.numpy as jnp
from jax import lax
from jax.experimental import pallas as pl
from jax.experimental.pallas import tpu as pltpu


def kernel(
    x,
):
    def body(*refs):
        pass

    out_shape = jax.ShapeDtypeStruct(..., jnp.float32)
    return pl.pallas_call(body, out_shape=out_shape)(...)


# baseline (device time: 32086 ns/iter reference)
import jax
import jax.numpy as jnp
from jax import lax
from jax.experimental import pallas as pl
from jax.experimental.pallas import tpu as pltpu

N_DEV = 4
N_HOPS = 2 * (N_DEV - 1)


def kernel(x):
    m, n = x.shape
    chunk = m // N_DEV

    def body(x_ref, out_ref, comm_ref, send_sems, recv_sems):
        my = lax.axis_index("i")
        left = (my - 1) % N_DEV
        right = (my + 1) % N_DEV

        barrier_sem = pltpu.get_barrier_semaphore()
        for nbr in (left, right):
            pl.semaphore_signal(
                barrier_sem, inc=1,
                device_id=(nbr,), device_id_type=pl.DeviceIdType.MESH,
            )
        pl.semaphore_wait(barrier_sem, 2)

        out_ref[:, :] = x_ref[:, :]

        for s in range(N_DEV - 1):
            send_c = (my - s) % N_DEV
            recv_c = (my - s - 1) % N_DEV
            rdma = pltpu.make_async_remote_copy(
                src_ref=out_ref.at[pl.ds(send_c * chunk, chunk), :],
                dst_ref=comm_ref.at[s],
                send_sem=send_sems.at[s],
                recv_sem=recv_sems.at[s],
                device_id=(right,),
                device_id_type=pl.DeviceIdType.MESH,
            )
            rdma.start()
            rdma.wait()
            rows = pl.ds(recv_c * chunk, chunk)
            out_ref[rows, :] = out_ref[rows, :] + comm_ref[s]

        for t in range(N_DEV - 1):
            h = (N_DEV - 1) + t
            send_c = (my + 1 - t) % N_DEV
            recv_c = (my - t) % N_DEV
            rdma = pltpu.make_async_remote_copy(
                src_ref=out_ref.at[pl.ds(send_c * chunk, chunk), :],
                dst_ref=comm_ref.at[h],
                send_sem=send_sems.at[h],
                recv_sem=recv_sems.at[h],
                device_id=(right,),
                device_id_type=pl.DeviceIdType.MESH,
            )
            rdma.start()
            rdma.wait()
            out_ref[pl.ds(recv_c * chunk, chunk), :] = comm_ref[h]

    return pl.pallas_call(
        body,
        out_shape=jax.ShapeDtypeStruct((m, n), x.dtype),
        in_specs=[pl.BlockSpec(memory_space=pltpu.VMEM)],
        out_specs=pl.BlockSpec(memory_space=pltpu.VMEM),
        scratch_shapes=[
            pltpu.VMEM((N_HOPS, chunk, n), x.dtype),
            pltpu.SemaphoreType.DMA((N_HOPS,)),
            pltpu.SemaphoreType.DMA((N_HOPS,)),
        ],
        compiler_params=pltpu.CompilerParams(collective_id=0),
    )(x)


# device time: 20318 ns/iter; 1.5792x vs baseline; 1.5792x over previous
import jax
import jax.numpy as jnp
from jax import lax
from jax.experimental import pallas as pl
from jax.experimental.pallas import tpu as pltpu


def kernel(x):
    m, n = x.shape
    hm = m // 2
    qm = m // 4
    hn = n // 2

    def body(x_ref, out_ref, b1, b2, send_sems, recv_sems):
        my = lax.axis_index("i")
        a = my // 2
        b = (my % 2) ^ a
        px = my ^ 1
        py = 3 - my

        barrier_sem = pltpu.get_barrier_semaphore()
        for nbr in (px, py):
            pl.semaphore_signal(
                barrier_sem, inc=1,
                device_id=(nbr,), device_id_type=pl.DeviceIdType.MESH,
            )
        pl.semaphore_wait(barrier_sem, 2)

        r1 = b * hm
        r2 = r1 + a * qm
        s1 = (1 - b) * hm
        s2 = r1 + (1 - a) * qm
        r1R = a * hm
        r2R = r1R + b * qm
        s1R = (1 - a) * hm
        s2R = r1R + (1 - b) * qm

        cL = pl.ds(0, hn)
        cR = pl.ds(hn, hn)

        def start(idx, target, src, dst):
            rdma = pltpu.make_async_remote_copy(
                src_ref=src, dst_ref=dst,
                send_sem=send_sems.at[idx], recv_sem=recv_sems.at[idx],
                device_id=(target,), device_id_type=pl.DeviceIdType.MESH,
            )
            rdma.start()
            return rdma

        def finish(*rdmas):
            for r in rdmas:
                r.wait_recv()
            for r in rdmas:
                r.wait_send()

        l = start(0, px, x_ref.at[pl.ds(s1, hm), cL], b1.at[0])
        r = start(1, py, x_ref.at[pl.ds(s1R, hm), cR], b1.at[1])
        finish(l, r)
        out_ref[pl.ds(r1, hm), cL] = x_ref[pl.ds(r1, hm), cL] + b1[0]
        out_ref[pl.ds(r1R, hm), cR] = x_ref[pl.ds(r1R, hm), cR] + b1[1]

        l = start(2, py, out_ref.at[pl.ds(s2, qm), cL], b2.at[0])
        r = start(3, px, out_ref.at[pl.ds(s2R, qm), cR], b2.at[1])
        finish(l, r)
        out_ref[pl.ds(r2, qm), cL] = out_ref[pl.ds(r2, qm), cL] + b2[0]
        out_ref[pl.ds(r2R, qm), cR] = out_ref[pl.ds(r2R, qm), cR] + b2[1]

        l = start(4, py, out_ref.at[pl.ds(r2, qm), cL],
                  out_ref.at[pl.ds(r2, qm), cL])
        r = start(5, px, out_ref.at[pl.ds(r2R, qm), cR],
                  out_ref.at[pl.ds(r2R, qm), cR])
        finish(l, r)

        l = start(6, px, out_ref.at[pl.ds(r1, hm), cL],
                  out_ref.at[pl.ds(r1, hm), cL])
        r = start(7, py, out_ref.at[pl.ds(r1R, hm), cR],
                  out_ref.at[pl.ds(r1R, hm), cR])
        finish(l, r)

    return pl.pallas_call(
        body,
        out_shape=jax.ShapeDtypeStruct((m, n), x.dtype),
        in_specs=[pl.BlockSpec(memory_space=pltpu.VMEM)],
        out_specs=pl.BlockSpec(memory_space=pltpu.VMEM),
        scratch_shapes=[
            pltpu.VMEM((2, hm, hn), x.dtype),
            pltpu.VMEM((2, qm, hn), x.dtype),
            pltpu.SemaphoreType.DMA((8,)),
            pltpu.SemaphoreType.DMA((8,)),
        ],
        compiler_params=pltpu.CompilerParams(collective_id=0),
    )(x)


# device time: 17869 ns/iter; 1.7956x vs baseline; 1.1371x over previous
import jax
import jax.numpy as jnp
from jax import lax
from jax.experimental import pallas as pl
from jax.experimental.pallas import tpu as pltpu


def kernel(x):
    m, n = x.shape
    hm = m // 2
    qm = m // 4
    hn = n // 2

    def body(x_ref, out_ref, b1, b2, send_sems, recv_sems):
        my = lax.axis_index("i")
        a = my // 2
        b = (my % 2) ^ a
        px = my ^ 1
        py = 3 - my

        barrier_sem = pltpu.get_barrier_semaphore()
        for nbr in (px, py):
            pl.semaphore_signal(
                barrier_sem, inc=1,
                device_id=(nbr,), device_id_type=pl.DeviceIdType.MESH,
            )
        pl.semaphore_wait(barrier_sem, 2)

        r1 = b * hm
        r2 = r1 + a * qm
        s1 = (1 - b) * hm
        s2 = r1 + (1 - a) * qm
        r1R = a * hm
        r2R = r1R + b * qm
        s1R = (1 - a) * hm
        s2R = r1R + (1 - b) * qm

        cL = pl.ds(0, hn)
        cR = pl.ds(hn, hn)

        sends = []

        def start(idx, target, src, dst):
            rdma = pltpu.make_async_remote_copy(
                src_ref=src, dst_ref=dst,
                send_sem=send_sems.at[idx], recv_sem=recv_sems.at[idx],
                device_id=(target,), device_id_type=pl.DeviceIdType.MESH,
            )
            rdma.start()
            sends.append(rdma)
            return rdma

        p1aL = start(0, px, x_ref.at[pl.ds(s1 + (1 - a) * qm, qm), cL],
                     b1.at[0, pl.ds((1 - a) * qm, qm), :])
        p1aR = start(1, py, x_ref.at[pl.ds(s1R + (1 - b) * qm, qm), cR],
                     b1.at[1, pl.ds((1 - b) * qm, qm), :])
        p1bL = start(2, px, x_ref.at[pl.ds(s1 + a * qm, qm), cL],
                     b1.at[0, pl.ds(a * qm, qm), :])
        p1bR = start(3, py, x_ref.at[pl.ds(s1R + b * qm, qm), cR],
                     b1.at[1, pl.ds(b * qm, qm), :])

        p1aL.wait_recv()
        out_ref[pl.ds(s2, qm), cL] = (
            x_ref[pl.ds(s2, qm), cL] + b1[0, pl.ds((1 - a) * qm, qm), :]
        )
        p2L = start(4, py, out_ref.at[pl.ds(s2, qm), cL], b2.at[0])
        p1aR.wait_recv()
        out_ref[pl.ds(s2R, qm), cR] = (
            x_ref[pl.ds(s2R, qm), cR] + b1[1, pl.ds((1 - b) * qm, qm), :]
        )
        p2R = start(5, px, out_ref.at[pl.ds(s2R, qm), cR], b2.at[1])

        p1bL.wait_recv()
        out_ref[pl.ds(r2, qm), cL] = (
            x_ref[pl.ds(r2, qm), cL] + b1[0, pl.ds(a * qm, qm), :]
        )
        p1bR.wait_recv()
        out_ref[pl.ds(r2R, qm), cR] = (
            x_ref[pl.ds(r2R, qm), cR] + b1[1, pl.ds(b * qm, qm), :]
        )

        p2L.wait_recv()
        out_ref[pl.ds(r2, qm), cL] = out_ref[pl.ds(r2, qm), cL] + b2[0]
        p3L = start(6, py, out_ref.at[pl.ds(r2, qm), cL],
                    out_ref.at[pl.ds(r2, qm), cL])
        p4aL = start(8, px, out_ref.at[pl.ds(r2, qm), cL],
                     out_ref.at[pl.ds(r2, qm), cL])
        p2R.wait_recv()
        out_ref[pl.ds(r2R, qm), cR] = out_ref[pl.ds(r2R, qm), cR] + b2[1]
        p3R = start(7, px, out_ref.at[pl.ds(r2R, qm), cR],
                    out_ref.at[pl.ds(r2R, qm), cR])
        p4aR = start(9, py, out_ref.at[pl.ds(r2R, qm), cR],
                     out_ref.at[pl.ds(r2R, qm), cR])

        p3L.wait_recv()
        p4bL = start(10, px, out_ref.at[pl.ds(s2, qm), cL],
                     out_ref.at[pl.ds(s2, qm), cL])
        p3R.wait_recv()
        p4bR = start(11, py, out_ref.at[pl.ds(s2R, qm), cR],
                     out_ref.at[pl.ds(s2R, qm), cR])

        p4aL.wait_recv()
        p4aR.wait_recv()
        p4bL.wait_recv()
        p4bR.wait_recv()

        for rdma in sends:
            rdma.wait_send()

    return pl.pallas_call(
        body,
        out_shape=jax.ShapeDtypeStruct((m, n), x.dtype),
        in_specs=[pl.BlockSpec(memory_space=pltpu.VMEM)],
        out_specs=pl.BlockSpec(memory_space=pltpu.VMEM),
        scratch_shapes=[
            pltpu.VMEM((2, hm, hn), x.dtype),
            pltpu.VMEM((2, qm, hn), x.dtype),
            pltpu.SemaphoreType.DMA((12,)),
            pltpu.SemaphoreType.DMA((12,)),
        ],
        compiler_params=pltpu.CompilerParams(collective_id=0),
    )(x)


# device time: 17666 ns/iter; 1.8163x vs baseline; 1.0115x over previous
import jax
import jax.numpy as jnp
from jax import lax
from jax.experimental import pallas as pl
from jax.experimental.pallas import tpu as pltpu


def kernel(x):
    m, n = x.shape
    hm = m // 2
    qm = m // 4
    hn = n // 2

    def body(x_ref, out_ref, b1, b2, send_sems, recv_sems):
        my = lax.axis_index("i")
        a = my // 2
        b = (my % 2) ^ a
        px = my ^ 1
        py = 3 - my

        barrier_sem = pltpu.get_barrier_semaphore()
        for nbr in (px, py):
            pl.semaphore_signal(
                barrier_sem, inc=1,
                device_id=(nbr,), device_id_type=pl.DeviceIdType.MESH,
            )
        pl.semaphore_wait(barrier_sem, 2)

        r1 = b * hm
        r2 = r1 + a * qm
        s1 = (1 - b) * hm
        s2 = r1 + (1 - a) * qm
        r1R = a * hm
        r2R = r1R + b * qm
        s1R = (1 - a) * hm
        s2R = r1R + (1 - b) * qm

        cL = pl.ds(0, hn)
        cR = pl.ds(hn, hn)

        sends = []

        def start(idx, target, src, dst):
            rdma = pltpu.make_async_remote_copy(
                src_ref=src, dst_ref=dst,
                send_sem=send_sems.at[idx], recv_sem=recv_sems.at[idx],
                device_id=(target,), device_id_type=pl.DeviceIdType.MESH,
            )
            rdma.start()
            sends.append(rdma)
            return rdma

        p1aL = start(0, px, x_ref.at[pl.ds(s1 + (1 - a) * qm, qm), cL],
                     b1.at[0, pl.ds((1 - a) * qm, qm), :])
        p1aR = start(1, py, x_ref.at[pl.ds(s1R + (1 - b) * qm, qm), cR],
                     b1.at[1, pl.ds((1 - b) * qm, qm), :])
        p1bL = start(2, px, x_ref.at[pl.ds(s1 + a * qm, qm), cL],
                     b1.at[0, pl.ds(a * qm, qm), :])
        p1bR = start(3, py, x_ref.at[pl.ds(s1R + b * qm, qm), cR],
                     b1.at[1, pl.ds(b * qm, qm), :])

        p1aL.wait_recv()
        out_ref[pl.ds(s2, qm), cL] = (
            x_ref[pl.ds(s2, qm), cL] + b1[0, pl.ds((1 - a) * qm, qm), :]
        )
        p2L = start(4, py, out_ref.at[pl.ds(s2, qm), cL], b2.at[0])
        p1aR.wait_recv()
        out_ref[pl.ds(s2R, qm), cR] = (
            x_ref[pl.ds(s2R, qm), cR] + b1[1, pl.ds((1 - b) * qm, qm), :]
        )
        p2R = start(5, px, out_ref.at[pl.ds(s2R, qm), cR], b2.at[1])

        p1bL.wait_recv()
        p2L.wait_recv()
        out_ref[pl.ds(r2, qm), cL] = (
            x_ref[pl.ds(r2, qm), cL] + b1[0, pl.ds(a * qm, qm), :] + b2[0]
        )
        p3L = start(6, py, out_ref.at[pl.ds(r2, qm), cL],
                    out_ref.at[pl.ds(r2, qm), cL])
        p1bR.wait_recv()
        p2R.wait_recv()
        out_ref[pl.ds(r2R, qm), cR] = (
            x_ref[pl.ds(r2R, qm), cR] + b1[1, pl.ds(b * qm, qm), :] + b2[1]
        )
        p3R = start(7, px, out_ref.at[pl.ds(r2R, qm), cR],
                    out_ref.at[pl.ds(r2R, qm), cR])
        p4aL = start(8, px, out_ref.at[pl.ds(r2, qm), cL],
                     out_ref.at[pl.ds(r2, qm), cL])
        p4aR = start(9, py, out_ref.at[pl.ds(r2R, qm), cR],
                     out_ref.at[pl.ds(r2R, qm), cR])

        p3L.wait_recv()
        p4bL = start(10, px, out_ref.at[pl.ds(s2, qm), cL],
                     out_ref.at[pl.ds(s2, qm), cL])
        p3R.wait_recv()
        p4bR = start(11, py, out_ref.at[pl.ds(s2R, qm), cR],
                     out_ref.at[pl.ds(s2R, qm), cR])

        p4aL.wait_recv()
        p4aR.wait_recv()
        p4bL.wait_recv()
        p4bR.wait_recv()

        for rdma in sends:
            rdma.wait_send()

    return pl.pallas_call(
        body,
        out_shape=jax.ShapeDtypeStruct((m, n), x.dtype),
        in_specs=[pl.BlockSpec(memory_space=pltpu.VMEM)],
        out_specs=pl.BlockSpec(memory_space=pltpu.VMEM),
        scratch_shapes=[
            pltpu.VMEM((2, hm, hn), x.dtype),
            pltpu.VMEM((2, qm, hn), x.dtype),
            pltpu.SemaphoreType.DMA((12,)),
            pltpu.SemaphoreType.DMA((12,)),
        ],
        compiler_params=pltpu.CompilerParams(collective_id=0),
    )(x)


# device time: 16461 ns/iter; 1.9492x vs baseline; 1.0732x over previous
import jax
import jax.numpy as jnp
from jax import lax
from jax.experimental import pallas as pl
from jax.experimental.pallas import tpu as pltpu


def kernel(x):
    m, n = x.shape
    hm = m // 2
    qm = m // 4
    em = m // 8
    hn = n // 2

    def body(x_ref, out_ref, b1, b2, send_sems, recv_sems):
        my = lax.axis_index("i")
        a = my // 2
        b = (my % 2) ^ a
        px = my ^ 1
        py = 3 - my

        barrier_sem = pltpu.get_barrier_semaphore()
        for nbr in (px, py):
            pl.semaphore_signal(
                barrier_sem, inc=1,
                device_id=(nbr,), device_id_type=pl.DeviceIdType.MESH,
            )
        pl.semaphore_wait(barrier_sem, 2)

        r1 = b * hm
        r2 = r1 + a * qm
        s1 = (1 - b) * hm
        s2 = r1 + (1 - a) * qm
        r1R = a * hm
        r2R = r1R + b * qm
        s1R = (1 - a) * hm
        s2R = r1R + (1 - b) * qm
        f1 = (1 - a) * qm
        f1R = (1 - b) * qm
        k1 = a * qm
        k1R = b * qm

        cL = pl.ds(0, hn)
        cR = pl.ds(hn, hn)

        sends = []

        def start(idx, target, src, dst):
            rdma = pltpu.make_async_remote_copy(
                src_ref=src, dst_ref=dst,
                send_sem=send_sems.at[idx], recv_sem=recv_sems.at[idx],
                device_id=(target,), device_id_type=pl.DeviceIdType.MESH,
            )
            rdma.start()
            sends.append(rdma)
            return rdma

        p1a = [None, None, None, None]
        for i in range(2):
            p1a[2 * i] = start(
                0 + 2 * i, px,
                x_ref.at[pl.ds(s1 + f1 + i * em, em), cL],
                b1.at[0, pl.ds(f1 + i * em, em), :])
            p1a[2 * i + 1] = start(
                1 + 2 * i, py,
                x_ref.at[pl.ds(s1R + f1R + i * em, em), cR],
                b1.at[1, pl.ds(f1R + i * em, em), :])
        p1bL = start(4, px, x_ref.at[pl.ds(s1 + k1, qm), cL],
                     b1.at[0, pl.ds(k1, qm), :])
        p1bR = start(5, py, x_ref.at[pl.ds(s1R + k1R, qm), cR],
                     b1.at[1, pl.ds(k1R, qm), :])

        p2 = [None, None, None, None]
        for i in range(2):
            p1a[2 * i].wait_recv()
            rows = pl.ds(s2 + i * em, em)
            out_ref[rows, cL] = (
                x_ref[rows, cL] + b1[0, pl.ds(f1 + i * em, em), :]
            )
            p2[2 * i] = start(6 + 2 * i, py, out_ref.at[rows, cL],
                              b2.at[0, pl.ds(i * em, em), :])
            p1a[2 * i + 1].wait_recv()
            rowsR = pl.ds(s2R + i * em, em)
            out_ref[rowsR, cR] = (
                x_ref[rowsR, cR] + b1[1, pl.ds(f1R + i * em, em), :]
            )
            p2[2 * i + 1] = start(7 + 2 * i, px, out_ref.at[rowsR, cR],
                                  b2.at[1, pl.ds(i * em, em), :])

        p1bL.wait_recv()
        p1bR.wait_recv()
        p3 = [None, None, None, None]
        for i in range(2):
            p2[2 * i].wait_recv()
            rows = pl.ds(r2 + i * em, em)
            out_ref[rows, cL] = (
                x_ref[rows, cL]
                + b1[0, pl.ds(k1 + i * em, em), :]
                + b2[0, pl.ds(i * em, em), :]
            )
            p3[2 * i] = start(10 + 2 * i, py, out_ref.at[rows, cL],
                              out_ref.at[rows, cL])
            p2[2 * i + 1].wait_recv()
            rowsR = pl.ds(r2R + i * em, em)
            out_ref[rowsR, cR] = (
                x_ref[rowsR, cR]
                + b1[1, pl.ds(k1R + i * em, em), :]
                + b2[1, pl.ds(i * em, em), :]
            )
            p3[2 * i + 1] = start(11 + 2 * i, px, out_ref.at[rowsR, cR],
                                  out_ref.at[rowsR, cR])

        p4aL = start(14, px, out_ref.at[pl.ds(r2, qm), cL],
                     out_ref.at[pl.ds(r2, qm), cL])
        p4aR = start(15, py, out_ref.at[pl.ds(r2R, qm), cR],
                     out_ref.at[pl.ds(r2R, qm), cR])

        p4b = [None, None, None, None]
        for i in range(2):
            p3[2 * i].wait_recv()
            rows = pl.ds(s2 + i * em, em)
            p4b[2 * i] = start(16 + 2 * i, px, out_ref.at[rows, cL],
                               out_ref.at[rows, cL])
            p3[2 * i + 1].wait_recv()
            rowsR = pl.ds(s2R + i * em, em)
            p4b[2 * i + 1] = start(17 + 2 * i, py, out_ref.at[rowsR, cR],
                                   out_ref.at[rowsR, cR])

        p4aL.wait_recv()
        p4aR.wait_recv()
        for r in p4b:
            r.wait_recv()

        for rdma in sends:
            rdma.wait_send()

    return pl.pallas_call(
        body,
        out_shape=jax.ShapeDtypeStruct((m, n), x.dtype),
        in_specs=[pl.BlockSpec(memory_space=pltpu.VMEM)],
        out_specs=pl.BlockSpec(memory_space=pltpu.VMEM),
        scratch_shapes=[
            pltpu.VMEM((2, hm, hn), x.dtype),
            pltpu.VMEM((2, qm, hn), x.dtype),
            pltpu.SemaphoreType.DMA((20,)),
            pltpu.SemaphoreType.DMA((20,)),
        ],
        compiler_params=pltpu.CompilerParams(collective_id=0),
    )(x)


# device time: 16385 ns/iter; 1.9583x vs baseline; 1.0046x over previous
import jax
import jax.numpy as jnp
from jax import lax
from jax.experimental import pallas as pl
from jax.experimental.pallas import tpu as pltpu

S = 4


def kernel(x):
    m, n = x.shape
    hm = m // 2
    qm = m // 4
    em = qm // S
    hn = n // 2
    n_sems = 8 * S + 4

    def body(x_ref, out_ref, b1, b2, send_sems, recv_sems):
        my = lax.axis_index("i")
        a = my // 2
        b = (my % 2) ^ a
        px = my ^ 1
        py = 3 - my

        barrier_sem = pltpu.get_barrier_semaphore()
        for nbr in (px, py):
            pl.semaphore_signal(
                barrier_sem, inc=1,
                device_id=(nbr,), device_id_type=pl.DeviceIdType.MESH,
            )
        pl.semaphore_wait(barrier_sem, 2)

        r1 = b * hm
        r2 = r1 + a * qm
        s1 = (1 - b) * hm
        s2 = r1 + (1 - a) * qm
        r1R = a * hm
        r2R = r1R + b * qm
        s1R = (1 - a) * hm
        s2R = r1R + (1 - b) * qm
        f1 = (1 - a) * qm
        f1R = (1 - b) * qm
        k1 = a * qm
        k1R = b * qm

        cL = pl.ds(0, hn)
        cR = pl.ds(hn, hn)

        sends = []
        sem_idx = [0]

        def start(target, src, dst):
            idx = sem_idx[0]
            sem_idx[0] += 1
            rdma = pltpu.make_async_remote_copy(
                src_ref=src, dst_ref=dst,
                send_sem=send_sems.at[idx], recv_sem=recv_sems.at[idx],
                device_id=(target,), device_id_type=pl.DeviceIdType.MESH,
            )
            rdma.start()
            sends.append(rdma)
            return rdma

        p1aL, p1aR = [], []
        for i in range(S):
            p1aL.append(start(
                px, x_ref.at[pl.ds(s1 + f1 + i * em, em), cL],
                b1.at[0, pl.ds(f1 + i * em, em), :]))
            p1aR.append(start(
                py, x_ref.at[pl.ds(s1R + f1R + i * em, em), cR],
                b1.at[1, pl.ds(f1R + i * em, em), :]))
        p1bL = start(px, x_ref.at[pl.ds(s1 + k1, qm), cL],
                     b1.at[0, pl.ds(k1, qm), :])
        p1bR = start(py, x_ref.at[pl.ds(s1R + k1R, qm), cR],
                     b1.at[1, pl.ds(k1R, qm), :])

        p2L, p2R = [], []
        for i in range(S):
            p1aL[i].wait_recv()
            rows = pl.ds(s2 + i * em, em)
            out_ref[rows, cL] = (
                x_ref[rows, cL] + b1[0, pl.ds(f1 + i * em, em), :]
            )
            p2L.append(start(py, out_ref.at[rows, cL],
                             b2.at[0, pl.ds(i * em, em), :]))
            p1aR[i].wait_recv()
            rowsR = pl.ds(s2R + i * em, em)
            out_ref[rowsR, cR] = (
                x_ref[rowsR, cR] + b1[1, pl.ds(f1R + i * em, em), :]
            )
            p2R.append(start(px, out_ref.at[rowsR, cR],
                             b2.at[1, pl.ds(i * em, em), :]))

        p1bL.wait_recv()
        p1bR.wait_recv()
        p3L, p3R = [], []
        for i in range(S):
            p2L[i].wait_recv()
            rows = pl.ds(r2 + i * em, em)
            out_ref[rows, cL] = (
                x_ref[rows, cL]
                + b1[0, pl.ds(k1 + i * em, em), :]
                + b2[0, pl.ds(i * em, em), :]
            )
            p3L.append(start(py, out_ref.at[rows, cL],
                             out_ref.at[rows, cL]))
            p2R[i].wait_recv()
            rowsR = pl.ds(r2R + i * em, em)
            out_ref[rowsR, cR] = (
                x_ref[rowsR, cR]
                + b1[1, pl.ds(k1R + i * em, em), :]
                + b2[1, pl.ds(i * em, em), :]
            )
            p3R.append(start(px, out_ref.at[rowsR, cR],
                             out_ref.at[rowsR, cR]))

        p4aL = start(px, out_ref.at[pl.ds(r2, qm), cL],
                     out_ref.at[pl.ds(r2, qm), cL])
        p4aR = start(py, out_ref.at[pl.ds(r2R, qm), cR],
                     out_ref.at[pl.ds(r2R, qm), cR])

        p4b = []
        for i in range(S):
            p3L[i].wait_recv()
            rows = pl.ds(s2 + i * em, em)
            p4b.append(start(px, out_ref.at[rows, cL],
                             out_ref.at[rows, cL]))
            p3R[i].wait_recv()
            rowsR = pl.ds(s2R + i * em, em)
            p4b.append(start(py, out_ref.at[rowsR, cR],
                             out_ref.at[rowsR, cR]))

        p4aL.wait_recv()
        p4aR.wait_recv()
        for r in p4b:
            r.wait_recv()

        for rdma in sends:
            rdma.wait_send()

    return pl.pallas_call(
        body,
        out_shape=jax.ShapeDtypeStruct((m, n), x.dtype),
        in_specs=[pl.BlockSpec(memory_space=pltpu.VMEM)],
        out_specs=pl.BlockSpec(memory_space=pltpu.VMEM),
        scratch_shapes=[
            pltpu.VMEM((2, hm, hn), x.dtype),
            pltpu.VMEM((2, qm, hn), x.dtype),
            pltpu.SemaphoreType.DMA((n_sems,)),
            pltpu.SemaphoreType.DMA((n_sems,)),
        ],
        compiler_params=pltpu.CompilerParams(collective_id=0),
    )(x)
